# baseline (speedup 1.0000x reference)
import functools

import jax
import jax.numpy as jnp
from jax import lax
from jax.experimental import pallas as pl
from jax.experimental.pallas import tpu as pltpu


def _stream_kernel(sig_ref, *refs, q, sl, l, half, ns):
    j_refs, out_ref = refs[:ns], refs[ns]
    qq = q * q
    c = pl.program_id(0)
    j = pl.program_id(1)
    m = c * half + j

    row = lax.broadcasted_iota(jnp.int32, (qq, l), 0)
    match = sig_ref[...] == row - (row // q) * q

    cols = []
    ssq = jnp.float32(0.0)
    for h, jref in enumerate(j_refs):
        jb = jref[...]
        for k in range(sl // ns):
            sli = jb[:, k * l:(k + 1) * l]
            rk = m * sl + h * (sl // ns) + k
            lane = lax.broadcasted_iota(jnp.int32, sli.shape, 1)
            keep = jnp.logical_and(match, lane != rk)
            cols.append(jnp.sum(jnp.where(keep, sli, 0.0), axis=1,
                                keepdims=True))
        ssq = ssq + jnp.sum(jb * jb)
    msums = jnp.concatenate(cols, axis=1)

    rows = msums.T
    srow = lax.broadcasted_iota(jnp.int32, (sl, 1), 0)
    ssq_col = jnp.where(srow == 0, ssq, 0.0)
    out_ref[...] = jnp.concatenate([rows, ssq_col], axis=1)


def _finish_kernel(gt_ref, h_ref, rt_ref, sr_ref, wb_ref, out_ref,
                   *, q, lambda_h, lambda_j):
    qq = q * q
    gt = gt_ref[...]
    h = h_ref[...]
    L = h.shape[1]

    srow = lax.broadcasted_iota(jnp.int32, (qq + 1, q), 0)
    slane = lax.broadcasted_iota(jnp.int32, (qq + 1, q), 1)
    selt = jnp.where(jnp.logical_and(srow // q == slane, srow < qq), 1.0, 0.0)
    g_t = lax.dot_general(gt, selt, (((1,), (0,)), ((), ())),
                          precision=lax.Precision.HIGHEST,
                          preferred_element_type=jnp.float32)
    lane = lax.broadcasted_iota(jnp.int32, gt.shape, 1)
    ssq_j = jnp.sum(jnp.where(lane == qq, gt, 0.0))
    reg = lambda_h * jnp.sum(h * h) + lambda_j * ssq_j

    e_all = h.T + g_t
    p_t = jnp.where(lax.broadcasted_iota(jnp.int32, (L, rt_ref.shape[1]), 0)
                    == rt_ref[...], 1.0, 0.0)
    e2 = lax.dot_general(p_t, e_all, (((0,), (0,)), ((), ())),
                         precision=lax.Precision.HIGHEST,
                         preferred_element_type=jnp.float32)
    R = e2.shape[0]
    m = jnp.max(e2, axis=1, keepdims=True)
    lse = m + jnp.log(jnp.sum(jnp.exp(e2 - m), axis=1, keepdims=True))
    sr_col = sr_ref[...].T
    q_hot = lax.broadcasted_iota(jnp.int32, (R, q), 1) == sr_col
    e_sel = jnp.sum(jnp.where(q_hot, e2, 0.0), axis=1, keepdims=True)
    loss = (lse - e_sel) * wb_ref[...].T + reg
    out_ref[...] = loss.T


def kernel(H, J, sigma_i, r_tab, sigma_r_tab, w_b_tab):
    q, L = H.shape
    qq = q * q
    R = int(r_tab.shape[0])
    lambda_h = 0.01
    lambda_j = 0.01
    sl = 16 if L % 32 == 0 else 2
    ns = 16 if L % 32 == 0 else 2
    nb = L // sl
    half = nb // 2

    Hf = jnp.asarray(H, jnp.float32)
    Jf = jnp.asarray(J, jnp.float32)
    sig = jnp.asarray(sigma_i, jnp.int32).reshape(1, L)
    rt = jnp.asarray(r_tab, jnp.int32).reshape(1, R)
    sr = jnp.asarray(sigma_r_tab, jnp.int32).reshape(1, R)
    wb = jnp.asarray(w_b_tab, jnp.float32).reshape(1, R)

    def _j_spec(s):
        return pl.BlockSpec((qq, sl * L // ns),
                            lambda c, j, s=s: (0, ns * (c * half + j) + s))

    gt = pl.pallas_call(
        functools.partial(_stream_kernel, q=q, sl=sl, l=L, half=half, ns=ns),
        out_shape=jax.ShapeDtypeStruct((L, qq + 1), jnp.float32),
        grid=(2, half),
        in_specs=[pl.BlockSpec((1, L), lambda c, j: (0, 0))]
                 + [_j_spec(s) for s in range(ns)],
        out_specs=pl.BlockSpec((sl, qq + 1), lambda c, j: (c * half + j, 0)),
        compiler_params=pltpu.CompilerParams(
            dimension_semantics=("parallel", "arbitrary")),
    )(sig, *([Jf] * ns))

    out = pl.pallas_call(
        functools.partial(_finish_kernel, q=q,
                          lambda_h=lambda_h, lambda_j=lambda_j),
        out_shape=jax.ShapeDtypeStruct((1, R), jnp.float32),
    )(gt, Hf, rt, sr, wb)
    return out.reshape(R)


# speedup vs baseline: 1.1549x; 1.1549x over previous
import functools

import jax
import jax.numpy as jnp
from jax import lax
from jax.experimental import pallas as pl
from jax.experimental.pallas import tpu as pltpu


def _potts_kernel(sig_ref, h_ref, rt_ref, sr_ref, wb_ref, *refs,
                  q, sl, l, nb, ns, lambda_h, lambda_j):
    j_refs, out_ref, gt_ref = refs[:ns], refs[ns], refs[ns + 1]
    qq = q * q
    g = pl.program_id(0)

    row = lax.broadcasted_iota(jnp.int32, (qq, l), 0)
    match = sig_ref[...] == row - (row // q) * q

    cols = []
    ssq = jnp.float32(0.0)
    for h, jref in enumerate(j_refs):
        jb = jref[...]
        for k in range(sl // ns):
            sli = jb[:, k * l:(k + 1) * l]
            rk = g * sl + h * (sl // ns) + k
            lane = lax.broadcasted_iota(jnp.int32, sli.shape, 1)
            keep = jnp.logical_and(match, lane != rk)
            cols.append(jnp.sum(jnp.where(keep, sli, 0.0), axis=1,
                                keepdims=True))
        ssq = ssq + jnp.sum(jb * jb)
    msums = jnp.concatenate(cols, axis=1)

    rows = msums.T
    srow = lax.broadcasted_iota(jnp.int32, (sl, 1), 0)
    ssq_col = jnp.where(srow == 0, ssq, 0.0)
    gt_ref[pl.ds(g * sl, sl), :] = jnp.concatenate([rows, ssq_col], axis=1)

    @pl.when(g == nb - 1)
    def _finish():
        gt = gt_ref[...]
        hm = h_ref[...]

        sr2 = lax.broadcasted_iota(jnp.int32, (qq + 1, q), 0)
        sl2 = lax.broadcasted_iota(jnp.int32, (qq + 1, q), 1)
        selt = jnp.where(jnp.logical_and(sr2 // q == sl2, sr2 < qq), 1.0, 0.0)
        g_t = lax.dot_general(gt, selt, (((1,), (0,)), ((), ())),
                              precision=lax.Precision.HIGHEST,
                              preferred_element_type=jnp.float32)
        lane2 = lax.broadcasted_iota(jnp.int32, gt.shape, 1)
        ssq_j = jnp.sum(jnp.where(lane2 == qq, gt, 0.0))
        reg = lambda_h * jnp.sum(hm * hm) + lambda_j * ssq_j

        e_all = hm.T + g_t
        p_t = jnp.where(
            lax.broadcasted_iota(jnp.int32, (l, rt_ref.shape[1]), 0)
            == rt_ref[...], 1.0, 0.0)
        e2 = lax.dot_general(p_t, e_all, (((0,), (0,)), ((), ())),
                             precision=lax.Precision.HIGHEST,
                             preferred_element_type=jnp.float32)
        R = e2.shape[0]
        m = jnp.max(e2, axis=1, keepdims=True)
        lse = m + jnp.log(jnp.sum(jnp.exp(e2 - m), axis=1, keepdims=True))
        sr_col = sr_ref[...].T
        q_hot = lax.broadcasted_iota(jnp.int32, (R, q), 1) == sr_col
        e_sel = jnp.sum(jnp.where(q_hot, e2, 0.0), axis=1, keepdims=True)
        loss = (lse - e_sel) * wb_ref[...].T + reg
        out_ref[...] = loss.T


def kernel(H, J, sigma_i, r_tab, sigma_r_tab, w_b_tab):
    q, L = H.shape
    qq = q * q
    R = int(r_tab.shape[0])
    lambda_h = 0.01
    lambda_j = 0.01
    sl = 32 if L % 64 == 0 else 2
    ns = 16 if L % 64 == 0 else 2
    nb = L // sl

    Hf = jnp.asarray(H, jnp.float32)
    Jf = jnp.asarray(J, jnp.float32)
    sig = jnp.asarray(sigma_i, jnp.int32).reshape(1, L)
    rt = jnp.asarray(r_tab, jnp.int32).reshape(1, R)
    sr = jnp.asarray(sigma_r_tab, jnp.int32).reshape(1, R)
    wb = jnp.asarray(w_b_tab, jnp.float32).reshape(1, R)

    def _resident(shape):
        return pl.BlockSpec(shape, lambda g, shape=shape: tuple(0 for _ in shape))

    def _j_spec(s):
        return pl.BlockSpec((qq, sl * L // ns), lambda g, s=s: (0, ns * g + s))

    out = pl.pallas_call(
        functools.partial(_potts_kernel, q=q, sl=sl, l=L, nb=nb, ns=ns,
                          lambda_h=lambda_h, lambda_j=lambda_j),
        out_shape=jax.ShapeDtypeStruct((1, R), jnp.float32),
        grid=(nb,),
        in_specs=[_resident((1, L)), _resident((q, L)), _resident((1, R)),
                  _resident((1, R)), _resident((1, R))]
                 + [_j_spec(s) for s in range(ns)],
        out_specs=_resident((1, R)),
        scratch_shapes=[pltpu.VMEM((L, qq + 1), jnp.float32)],
        compiler_params=pltpu.CompilerParams(
            dimension_semantics=("arbitrary",)),
    )(sig, Hf, rt, sr, wb, *([Jf] * ns))
    return out.reshape(R)
